# Initial kernel scaffold; baseline (speedup 1.0000x reference)
import jax
import jax.numpy as jnp
from jax.experimental import pallas as pl


def kernel(pos, eps, batch, t):
    raise NotImplementedError("write your pallas kernel here")


# speedup vs baseline: 2.1216x; 2.1216x over previous
import functools

import jax
import jax.numpy as jnp
from jax import lax
from jax.experimental import pallas as pl
from jax.experimental.pallas import tpu as pltpu
from jax.experimental.pallas import tpu_sc as plsc

T_STEPS = 1000
EMB = 128
HALF = EMB // 2

NC = 2
NS = 16
NW = NC * NS

CHUNK = 128
INFLIGHT = 3
RING = 6


def _prep_body(t_ref, emb_ref, alpha_ref, sigma_ref):
    B = t_ref.shape[0]
    tf = t_ref[...].astype(jnp.float32)

    coli = lax.broadcasted_iota(jnp.int32, (B, EMB), 1)
    col = coli.astype(jnp.float32)
    fidx = jnp.where(col < HALF, col, col - HALF)
    freqs = jnp.exp(-jnp.log(10000.0) * fidx / HALF)
    args = tf * freqs
    emb_ref[...] = jnp.where(col < HALF, jnp.sin(args), jnp.cos(args))

    j = lax.broadcasted_iota(jnp.int32, (B, T_STEPS), 1).astype(jnp.float32)
    beta = 1e-4 + (0.02 - 1e-4) * j / (T_STEPS - 1)
    logs = jnp.where(j <= tf, jnp.log1p(-beta), 0.0)
    ac_t = jnp.exp(jnp.sum(logs, axis=1))
    alpha_ref[...] = jnp.sqrt(ac_t)
    sigma_ref[...] = jnp.sqrt(1.0 - ac_t)


def _mean_body(n_rows, n_blocks, epsf_ref, mean_ref):
    i = pl.program_id(0)
    blk = epsf_ref.shape[0]

    @pl.when(i == 0)
    def _():
        mean_ref[0, 0] = 0.0
        mean_ref[0, 1] = 0.0
        mean_ref[0, 2] = 0.0

    gi = i * blk + lax.broadcasted_iota(jnp.int32, (blk,), 0)
    comp = jnp.where(gi < 3 * n_rows, lax.rem(gi, 3), 3)
    x = epsf_ref[...]
    mean_ref[0, 0] += jnp.sum(jnp.where(comp == 0, x, 0.0))
    mean_ref[0, 1] += jnp.sum(jnp.where(comp == 1, x, 0.0))
    mean_ref[0, 2] += jnp.sum(jnp.where(comp == 2, x, 0.0))

    @pl.when(i == n_blocks - 1)
    def _():
        inv = 1.0 / n_rows
        mean_ref[0, 0] *= inv
        mean_ref[0, 1] *= inv
        mean_ref[0, 2] *= inv


def _sc_gather_body(n_rows, emb_hbm, alpha_hbm, sigma_hbm, batch_hbm,
                    cond_hbm, af_hbm, sf_hbm, *scratch):
    per_w = (n_rows + NW - 1) // NW
    per_w = ((per_w + CHUNK - 1) // CHUNK) * CHUNK
    nch = per_w // CHUNK

    idx_all = scratch[0]
    rows = scratch[1:1 + RING]
    afb = scratch[1 + RING:1 + 2 * RING]
    sfb = scratch[1 + 2 * RING:1 + 3 * RING]
    alpha_v = scratch[1 + 3 * RING]
    sigma_v = scratch[2 + 3 * RING]
    gsem = scratch[3 + 3 * RING:3 + 4 * RING]
    orsem = scratch[3 + 4 * RING:3 + 5 * RING]
    oasem = scratch[3 + 5 * RING:3 + 6 * RING]
    ossem = scratch[3 + 6 * RING:3 + 7 * RING]

    wid = lax.axis_index("s") * NC + lax.axis_index("c")
    base = jnp.minimum(wid * per_w, n_rows - per_w)

    pltpu.sync_copy(alpha_hbm, alpha_v)
    pltpu.sync_copy(sigma_hbm, sigma_v)
    pltpu.sync_copy(batch_hbm.at[pl.ds(base, per_w)], idx_all)

    lane = lax.broadcasted_iota(jnp.int32, (16,), 0)

    def start_gather(c):
        s = c % RING
        return pltpu.async_copy(
            emb_hbm.at[idx_all.at[pl.ds(c * CHUNK, CHUNK)]], rows[s], gsem[s])

    def build_asf(c):
        s = c % RING

        def k_body(k, carry):
            j = k * 16 + lane
            atom = (j * 21846) >> 16
            g = plsc.load_gather(idx_all, [c * CHUNK + atom])
            a = plsc.load_gather(alpha_v, [g])
            sg = plsc.load_gather(sigma_v, [g])
            afb[s][pl.ds(k * 16, 16)] = a
            sfb[s][pl.ds(k * 16, 16)] = sg
            return carry

        lax.fori_loop(0, 3 * CHUNK // 16, k_body, 0)

    pend = {}

    for p in range(min(INFLIGHT, nch)):
        pend[p] = [start_gather(p)]

    for c in range(nch):
        s = c % RING
        start = base + c * CHUNK
        cps = pend.pop(c)
        cps[0].wait()
        build_asf(c)
        out = [
            pltpu.async_copy(rows[s], cond_hbm.at[pl.ds(start, CHUNK)], orsem[s]),
            pltpu.async_copy(afb[s], af_hbm.at[pl.ds(start * 3, 3 * CHUNK)], oasem[s]),
            pltpu.async_copy(sfb[s], sf_hbm.at[pl.ds(start * 3, 3 * CHUNK)], ossem[s]),
        ]
        pend[c] = out
        nxt = c + INFLIGHT
        if nxt < nch:
            prev = nxt - RING
            if prev >= 0 and prev in pend:
                for cp in pend.pop(prev):
                    cp.wait()
            pend[nxt] = [start_gather(nxt)]

    for c in sorted(pend):
        for cp in pend[c]:
            cp.wait()


def _combine_body(posf_ref, epsf_ref, af_ref, sf_ref, mean_ref,
                  noisedf_ref, epscf_ref):
    i = pl.program_id(0)
    blk = posf_ref.shape[0]
    gi = i * blk + lax.broadcasted_iota(jnp.int32, (blk,), 0)
    comp = lax.rem(gi, 3)
    m0 = mean_ref[0, 0]
    m1 = mean_ref[0, 1]
    m2 = mean_ref[0, 2]
    meanv = jnp.where(comp == 0, m0, jnp.where(comp == 1, m1, m2))
    ec = epsf_ref[...] - meanv
    epscf_ref[...] = ec
    noisedf_ref[...] = af_ref[...] * posf_ref[...] + sf_ref[...] * ec


def kernel(pos, eps, batch, t):
    n, _ = pos.shape
    b = t.shape[0]
    nf = 3 * n

    posf = pos.reshape(nf)
    epsf = eps.reshape(nf)

    emb_table, alpha_t, sigma_t = pl.pallas_call(
        _prep_body,
        out_shape=(
            jax.ShapeDtypeStruct((b, EMB), jnp.float32),
            jax.ShapeDtypeStruct((b,), jnp.float32),
            jax.ShapeDtypeStruct((b,), jnp.float32),
        ),
    )(t)

    blk = 51200
    n_blocks = (nf + blk - 1) // blk
    mean = pl.pallas_call(
        functools.partial(_mean_body, n, n_blocks),
        grid=(n_blocks,),
        in_specs=[pl.BlockSpec((blk,), lambda i: (i,))],
        out_specs=pl.BlockSpec(memory_space=pltpu.SMEM),
        out_shape=jax.ShapeDtypeStruct((1, 3), jnp.float32),
    )(epsf)

    mesh = plsc.VectorSubcoreMesh(
        core_axis_name="c", subcore_axis_name="s", num_cores=NC, num_subcores=NS
    )
    per_w = ((n + NW * CHUNK - 1) // (NW * CHUNK)) * CHUNK
    conditioning, af, sf = pl.kernel(
        functools.partial(_sc_gather_body, n),
        out_type=(
            jax.ShapeDtypeStruct((n, EMB), jnp.float32),
            jax.ShapeDtypeStruct((nf,), jnp.float32),
            jax.ShapeDtypeStruct((nf,), jnp.float32),
        ),
        mesh=mesh,
        scratch_types=(
            [pltpu.VMEM((per_w,), jnp.int32)]
            + [pltpu.VMEM((CHUNK, EMB), jnp.float32) for _ in range(RING)]
            + [pltpu.VMEM((3 * CHUNK,), jnp.float32) for _ in range(RING)]
            + [pltpu.VMEM((3 * CHUNK,), jnp.float32) for _ in range(RING)]
            + [pltpu.VMEM((b,), jnp.float32), pltpu.VMEM((b,), jnp.float32)]
            + [pltpu.SemaphoreType.DMA for _ in range(4 * RING)]
        ),
        compiler_params=pltpu.CompilerParams(needs_layout_passes=False),
    )(emb_table, alpha_t, sigma_t, batch)

    noisedf, epscf = pl.pallas_call(
        _combine_body,
        grid=(n_blocks,),
        in_specs=[
            pl.BlockSpec((blk,), lambda i: (i,)),
            pl.BlockSpec((blk,), lambda i: (i,)),
            pl.BlockSpec((blk,), lambda i: (i,)),
            pl.BlockSpec((blk,), lambda i: (i,)),
            pl.BlockSpec(memory_space=pltpu.SMEM),
        ],
        out_specs=(
            pl.BlockSpec((blk,), lambda i: (i,)),
            pl.BlockSpec((blk,), lambda i: (i,)),
        ),
        out_shape=(
            jax.ShapeDtypeStruct((nf,), jnp.float32),
            jax.ShapeDtypeStruct((nf,), jnp.float32),
        ),
    )(posf, epsf, af, sf, mean)

    return noisedf.reshape(n, 3), epscf.reshape(n, 3), conditioning, t
